# Initial kernel scaffold; baseline (speedup 1.0000x reference)
import jax
import jax.numpy as jnp
from jax.experimental import pallas as pl


def kernel(edge_index, edge_cost, edge_time, fastest_time, params_p, params_p2, params_pp, node, dest, remaining_time):
    raise NotImplementedError("write your pallas kernel here")


# speedup vs baseline: 1.4966x; 1.4966x over previous
import jax
import jax.numpy as jnp
from jax import lax
from jax.experimental import pallas as pl
from jax.experimental.pallas import tpu as pltpu

N = 10000
DIM = 256


def _finish_body(coef_ref, pT_ref, p2T_ref, pp0_ref, pp1_ref, out_ref):
    r2 = jnp.maximum(pT_ref[2:3, :], 0.0)
    r3 = jnp.maximum(pT_ref[3:4, :], 0.0)
    dn = (((1,), (1,)), ((), ()))
    v0 = lax.dot_general(r2, pp0_ref[:, :], dn,
                         preferred_element_type=jnp.float32)
    v1 = lax.dot_general(r3, pp1_ref[:, :], dn,
                         preferred_element_type=jnp.float32)
    bmat = jnp.concatenate([
        pT_ref[4:5, :], pT_ref[5:6, :], v0, v1,
        p2T_ref[0:1, :], p2T_ref[1:2, :], p2T_ref[2:3, :], p2T_ref[3:4, :],
    ], axis=0)
    out3 = lax.dot_general(coef_ref[:, :], bmat, (((1,), (0,)), ((), ())),
                           preferred_element_type=jnp.float32)
    p1 = pT_ref[1:2, :]
    out_ref[:, :] = out3[0:1] + p1 * out3[1:2] + (p1 * p1) * out3[2:3]


def _finish(coef, params_p, params_p2, params_pp):
    pT = params_p.T
    p2T = jnp.stack([params_p2[:, 0, 0], params_p2[:, 1, 0],
                     params_p2[:, 0, 1], params_p2[:, 1, 1]], axis=0)
    pp0 = params_pp[:, :, 0]
    pp1 = params_pp[:, :, 1]
    out = pl.pallas_call(
        _finish_body,
        out_shape=jax.ShapeDtypeStruct((1, DIM), jnp.float32),
    )(coef, pT, p2T, pp0, pp1)
    return out[0]


def kernel(edge_index, edge_cost, edge_time, fastest_time, params_p,
           params_p2, params_pp, node, dest, remaining_time):
    n = fastest_time.shape[0]
    src = edge_index[0]
    dst = edge_index[1]
    rt0 = jnp.float32(remaining_time)

    ones = jnp.ones_like(edge_cost)
    deg = jax.ops.segment_sum(ones, src, num_segments=n)
    degs = jnp.maximum(deg, 1.0)
    meanc = jax.ops.segment_sum(edge_cost, src, num_segments=n) / degs
    meant = jax.ops.segment_sum(edge_time, src, num_segments=n) / degs

    minc = jax.ops.segment_min(edge_cost, src, num_segments=n)
    tminc = jax.ops.segment_min(
        jnp.where(edge_cost <= minc[src], edge_time, jnp.inf), src,
        num_segments=n)
    mint = jax.ops.segment_min(edge_time, src, num_segments=n)
    cmint = jax.ops.segment_min(
        jnp.where(edge_time <= mint[src], edge_cost, jnp.inf), src,
        num_segments=n)
    has = deg > 0
    minc = jnp.where(has, minc, 10000.0)
    tminc = jnp.where(has, tminc, 0.0)
    mint = jnp.where(has, mint, 10000.0)
    cmint = jnp.where(has, cmint, 0.0)

    m0 = (src == node).astype(jnp.float32)
    cnt1 = jax.ops.segment_sum(m0, dst, num_segments=n)
    sum1 = jax.ops.segment_sum(m0 * (rt0 - edge_time), dst, num_segments=n)
    trem1 = sum1 / jnp.maximum(cnt1, 1.0)
    reach1 = (cnt1 > 0).astype(jnp.float32)
    g1 = reach1[src]
    cnt2 = jax.ops.segment_sum(g1, dst, num_segments=n)
    sum2 = jax.ops.segment_sum(g1 * (trem1[src] - edge_time), dst,
                               num_segments=n)
    trem2 = sum2 / jnp.maximum(cnt2, 1.0)

    g2mat = jnp.stack([fastest_time, trem2, meanc, meant,
                       minc, tminc, cmint, mint], axis=1)
    a2 = jax.ops.segment_sum(g2mat[dst], src, num_segments=n) / degs[:, None]

    mask = (src == node).astype(jnp.float32)
    g1col = jnp.where(jnp.arange(8) == 1, trem1[dst, None],
                      g2mat[dst])
    dnode = degs[node]
    s1 = jnp.sum(mask[:, None] * g1col, axis=0) / dnode
    s2 = jnp.sum(mask[:, None] * a2[dst], axis=0) / dnode

    g0 = jnp.stack([fastest_time[node], rt0, meanc[node], meant[node],
                    minc[node], tminc[node], cmint[node], mint[node]])
    coef = jnp.stack([g0, s1, s2], axis=0)
    return _finish(coef, params_p, params_p2, params_pp)


# speedup vs baseline: 38.5265x; 25.7427x over previous
import jax
import jax.numpy as jnp
from jax import lax
from jax.experimental import pallas as pl
from jax.experimental.pallas import tpu as pltpu
from jax.experimental.pallas import tpu_sc as plsc

N = 10000
E = 160000
DIM = 256
NPAD = 10240
NT = 16
EPT = E // NT
RPT = NPAD // NT
NV = RPT // 16
NCHUNK = EPT // 16
BIG = 1e30
NQ = 10


def _fill(ref, nvec, val, dtype):
    vec = jnp.full((16,), val, dtype)

    def body(j, c):
        ref[pl.ds(j * 16, 16)] = vec
        return c

    lax.fori_loop(0, nvec, body, 0)


def _merge(mbref, rbase, bb, out_tmp, is_min):
    init = BIG if is_min else 0.0

    def _half(h, first):
        pltpu.sync_copy(mbref.at[pl.ds(h, 8), pl.ds(rbase, RPT)], bb)

        def body(v, c):
            acc = bb[0, pl.ds(v * 16, 16)]
            for r in range(1, 8):
                x = bb[r, pl.ds(v * 16, 16)]
                acc = jnp.minimum(acc, x) if is_min else acc + x
            if first:
                out_tmp[pl.ds(v * 16, 16)] = acc
            else:
                prev = out_tmp[pl.ds(v * 16, 16)]
                out_tmp[pl.ds(v * 16, 16)] = (
                    jnp.minimum(prev, acc) if is_min else prev + acc)
            return c

        lax.fori_loop(0, NV, body, 0)

    del init
    _half(0, True)
    _half(8, False)


def _sc_body(src_hbm, dst_hbm, cost_hbm, time_hbm, fast_hbm, node_hbm,
             rt_hbm, out_hbm,
             src_loc, dst_loc, cost_loc, time_loc,
             w0, w1, w2, w3, w4,
             bb, tmp1, tmp2, tmp3,
             qb, acc1b, acc2b, nv, rv, st3,
             mb0, mb1,
             q0, q1, q2, q3, q4, q5, q6, q7, q8, q9, q10, q11,
             accsh):
    sid = lax.axis_index("s")
    cid = lax.axis_index("c")
    ebase = sid * EPT
    rbase = sid * RPT

    pltpu.sync_copy(src_hbm.at[pl.ds(ebase, EPT)], src_loc)
    pltpu.sync_copy(dst_hbm.at[pl.ds(ebase, EPT)], dst_loc)
    pltpu.sync_copy(cost_hbm.at[pl.ds(ebase, EPT)], cost_loc)
    pltpu.sync_copy(time_hbm.at[pl.ds(ebase, EPT)], time_loc)
    pltpu.sync_copy(node_hbm, nv)
    pltpu.sync_copy(rt_hbm, rv)
    nodesc = nv[:][0]
    rt0 = rv[:][0]

    _fill(w0, NPAD // 16, 0.0, jnp.float32)
    _fill(w1, NPAD // 16, 0.0, jnp.float32)
    _fill(w2, NPAD // 16, 0.0, jnp.float32)
    _fill(w3, NPAD // 16, BIG, jnp.float32)
    _fill(w4, NPAD // 16, BIG, jnp.float32)
    ones16 = jnp.ones((16,), jnp.float32)

    def p1_body(i, c):
        b = i * 16
        srcv = src_loc[pl.ds(b, 16)]
        costv = cost_loc[pl.ds(b, 16)]
        timev = time_loc[pl.ds(b, 16)]
        cnt, _ = plsc.scan_count(srcv)
        cntf = cnt.astype(jnp.float32)
        mn = jnp.min(cntf).astype(jnp.int32)
        mx = jnp.max(cntf).astype(jnp.int32)

        def rbody(r, c2):
            rm = cnt == r
            plsc.addupdate_scatter(w0, [srcv], ones16, mask=rm)
            plsc.addupdate_scatter(w1, [srcv], costv, mask=rm)
            plsc.addupdate_scatter(w2, [srcv], timev, mask=rm)
            curc = plsc.load_gather(w3, [srcv])
            plsc.store_scatter(w3, [srcv], jnp.minimum(curc, costv), mask=rm)
            curt = plsc.load_gather(w4, [srcv])
            plsc.store_scatter(w4, [srcv], jnp.minimum(curt, timev), mask=rm)
            return c2

        lax.fori_loop(mn, mx + 1, rbody, 0)
        return c

    lax.fori_loop(0, NCHUNK, p1_body, 0)

    def ew(dst_ref, fn):
        def body(v, c):
            sl = pl.ds(v * 16, 16)
            dst_ref[sl] = fn(sl)
            return c

        lax.fori_loop(0, NV, body, 0)

    pltpu.sync_copy(w0, mb0.at[sid])
    pltpu.sync_copy(w1, mb1.at[sid])
    plsc.subcore_barrier()
    _merge(mb0, rbase, bb, tmp1, False)
    ew(tmp2, lambda sl: jnp.maximum(tmp1[sl], 1.0))
    pltpu.sync_copy(tmp1, q11.at[pl.ds(rbase, RPT)])
    pltpu.sync_copy(tmp2, q8.at[pl.ds(rbase, RPT)])
    _merge(mb1, rbase, bb, tmp3, False)
    ew(tmp3, lambda sl: tmp3[sl] / tmp2[sl])
    pltpu.sync_copy(tmp3, q2.at[pl.ds(rbase, RPT)])
    plsc.subcore_barrier()

    pltpu.sync_copy(w2, mb0.at[sid])
    pltpu.sync_copy(w3, mb1.at[sid])
    plsc.subcore_barrier()
    _merge(mb0, rbase, bb, tmp3, False)
    ew(tmp3, lambda sl: tmp3[sl] / tmp2[sl])
    pltpu.sync_copy(tmp3, q3.at[pl.ds(rbase, RPT)])
    _merge(mb1, rbase, bb, tmp3, True)
    ew(tmp3, lambda sl: jnp.where(tmp1[sl] > 0.0, tmp3[sl], 10000.0))
    pltpu.sync_copy(tmp3, q4.at[pl.ds(rbase, RPT)])
    plsc.subcore_barrier()

    pltpu.sync_copy(w4, mb0.at[sid])
    plsc.subcore_barrier()
    _merge(mb0, rbase, bb, tmp3, True)
    ew(tmp3, lambda sl: jnp.where(tmp1[sl] > 0.0, tmp3[sl], 10000.0))
    pltpu.sync_copy(tmp3, q7.at[pl.ds(rbase, RPT)])
    pltpu.sync_copy(fast_hbm.at[pl.ds(rbase, RPT)], tmp3)
    pltpu.sync_copy(tmp3, q0.at[pl.ds(rbase, RPT)])
    plsc.subcore_barrier()

    pltpu.sync_copy(q4, w0)
    pltpu.sync_copy(q7, w1)
    _fill(w2, NPAD // 16, BIG, jnp.float32)
    _fill(w3, NPAD // 16, BIG, jnp.float32)

    def p2_body(i, c):
        b = i * 16
        srcv = src_loc[pl.ds(b, 16)]
        costv = cost_loc[pl.ds(b, 16)]
        timev = time_loc[pl.ds(b, 16)]
        mc = plsc.load_gather(w0, [srcv])
        mt = plsc.load_gather(w1, [srcv])
        candt = jnp.where(costv <= mc, timev, BIG)
        candc = jnp.where(timev <= mt, costv, BIG)
        cnt, _ = plsc.scan_count(srcv)
        cntf = cnt.astype(jnp.float32)
        mn = jnp.min(cntf).astype(jnp.int32)
        mx = jnp.max(cntf).astype(jnp.int32)

        def rbody(r, c2):
            rm = cnt == r
            cur1 = plsc.load_gather(w2, [srcv])
            plsc.store_scatter(w2, [srcv], jnp.minimum(cur1, candt), mask=rm)
            cur2 = plsc.load_gather(w3, [srcv])
            plsc.store_scatter(w3, [srcv], jnp.minimum(cur2, candc), mask=rm)
            return c2

        lax.fori_loop(mn, mx + 1, rbody, 0)
        return c

    lax.fori_loop(0, NCHUNK, p2_body, 0)
    pltpu.sync_copy(w2, mb0.at[sid])
    pltpu.sync_copy(w3, mb1.at[sid])
    plsc.subcore_barrier()
    pltpu.sync_copy(q11.at[pl.ds(rbase, RPT)], tmp1)
    _merge(mb0, rbase, bb, tmp3, True)
    ew(tmp3, lambda sl: jnp.where(tmp1[sl] > 0.0, tmp3[sl], 0.0))
    pltpu.sync_copy(tmp3, q5.at[pl.ds(rbase, RPT)])
    _merge(mb1, rbase, bb, tmp3, True)
    ew(tmp3, lambda sl: jnp.where(tmp1[sl] > 0.0, tmp3[sl], 0.0))
    pltpu.sync_copy(tmp3, q6.at[pl.ds(rbase, RPT)])
    plsc.subcore_barrier()

    _fill(w0, NPAD // 16, 0.0, jnp.float32)
    _fill(w1, NPAD // 16, 0.0, jnp.float32)

    def p3_body(i, c):
        b = i * 16
        srcv = src_loc[pl.ds(b, 16)]
        m = srcv == nodesc
        anyv = jnp.max(jnp.where(m, 1.0, 0.0))

        @pl.when(anyv > 0.0)
        def _():
            dstv = dst_loc[pl.ds(b, 16)]
            timev = time_loc[pl.ds(b, 16)]
            mf = jnp.where(m, 1.0, 0.0)
            cnt, _ = plsc.scan_count(dstv)
            cntf = cnt.astype(jnp.float32)
            mn = jnp.min(cntf).astype(jnp.int32)
            mx = jnp.max(cntf).astype(jnp.int32)

            def rbody(r, c2):
                rm = cnt == r
                plsc.addupdate_scatter(w0, [dstv], mf, mask=rm)
                plsc.addupdate_scatter(w1, [dstv], mf * (rt0 - timev),
                                       mask=rm)
                return c2

            lax.fori_loop(mn, mx + 1, rbody, 0)

        return c

    lax.fori_loop(0, NCHUNK, p3_body, 0)
    pltpu.sync_copy(w0, mb0.at[sid])
    pltpu.sync_copy(w1, mb1.at[sid])
    plsc.subcore_barrier()
    pltpu.sync_copy(q8.at[pl.ds(rbase, RPT)], tmp2)
    _merge(mb0, rbase, bb, tmp1, False)
    ew(tmp3, lambda sl: tmp1[sl] / tmp2[sl])
    pltpu.sync_copy(tmp3, q10.at[pl.ds(rbase, RPT)])
    _merge(mb1, rbase, bb, tmp3, False)
    ew(tmp3, lambda sl: tmp3[sl] / jnp.maximum(tmp1[sl], 1.0))
    pltpu.sync_copy(tmp3, q9.at[pl.ds(rbase, RPT)])
    plsc.subcore_barrier()

    pltpu.sync_copy(q10, w2)
    pltpu.sync_copy(q9, w3)
    _fill(w0, NPAD // 16, 0.0, jnp.float32)
    _fill(w1, NPAD // 16, 0.0, jnp.float32)

    def p4_body(i, c):
        b = i * 16
        srcv = src_loc[pl.ds(b, 16)]
        wv = plsc.load_gather(w2, [srcv])
        anyv = jnp.max(wv)

        @pl.when(anyv > 0.0)
        def _():
            dstv = dst_loc[pl.ds(b, 16)]
            timev = time_loc[pl.ds(b, 16)]
            gf = jnp.where(wv > 0.0, 1.0, 0.0)
            t1 = plsc.load_gather(w3, [srcv])
            cnt, _ = plsc.scan_count(dstv)
            cntf = cnt.astype(jnp.float32)
            mn = jnp.min(cntf).astype(jnp.int32)
            mx = jnp.max(cntf).astype(jnp.int32)

            def rbody(r, c2):
                rm = cnt == r
                plsc.addupdate_scatter(w0, [dstv], gf, mask=rm)
                plsc.addupdate_scatter(w1, [dstv], gf * (t1 - timev),
                                       mask=rm)
                return c2

            lax.fori_loop(mn, mx + 1, rbody, 0)

        return c

    lax.fori_loop(0, NCHUNK, p4_body, 0)
    pltpu.sync_copy(w0, mb0.at[sid])
    pltpu.sync_copy(w1, mb1.at[sid])
    plsc.subcore_barrier()
    _merge(mb0, rbase, bb, tmp1, False)
    _merge(mb1, rbase, bb, tmp3, False)
    ew(tmp3, lambda sl: tmp3[sl] / jnp.maximum(tmp1[sl], 1.0))
    pltpu.sync_copy(tmp3, q1.at[pl.ds(rbase, RPT)])
    plsc.subcore_barrier()

    z16 = jnp.zeros((16,), jnp.float32)
    for k in range(NQ):
        acc1b[pl.ds(k * 16, 16)] = z16
        acc2b[pl.ds(k * 16, 16)] = z16
    qrefs = (q0, q1, q2, q3, q4, q5, q6, q7, q8, q9)

    def p5_body(i, c):
        b = i * 16
        srcv = src_loc[pl.ds(b, 16)]
        wv = plsc.load_gather(w2, [srcv])
        mbm = srcv == nodesc
        mbf = jnp.where(mbm, 1.0, 0.0)
        act = jnp.maximum(jnp.max(wv), jnp.max(mbf))

        @pl.when(act > 0.0)
        def _():
            dstv = dst_loc[pl.ds(b, 16)]
            for k in range(NQ):
                pltpu.sync_copy(qrefs[k].at[dstv],
                                qb.at[pl.ds(k * 16, 16)])
            for k in range(NQ):
                sl = pl.ds(k * 16, 16)
                qv = qb[sl]
                acc2b[sl] = acc2b[sl] + wv * qv
                acc1b[sl] = acc1b[sl] + mbf * qv

        return c

    lax.fori_loop(0, NCHUNK, p5_body, 0)
    pltpu.sync_copy(acc1b, accsh.at[pl.ds(sid * 2 * NQ * 16, NQ * 16)])
    pltpu.sync_copy(acc2b,
                    accsh.at[pl.ds(sid * 2 * NQ * 16 + NQ * 16, NQ * 16)])
    plsc.subcore_barrier()

    @pl.when(jnp.logical_and(sid == 0, cid == 0))
    def _():
        for k in range(NQ):
            acc1b[pl.ds(k * 16, 16)] = z16
            acc2b[pl.ds(k * 16, 16)] = z16
        for r in range(NT):
            pltpu.sync_copy(accsh.at[pl.ds(r * 2 * NQ * 16, NQ * 16)], qb)
            for k in range(NQ):
                sl = pl.ds(k * 16, 16)
                acc1b[sl] = acc1b[sl] + qb[sl]
            pltpu.sync_copy(
                accsh.at[pl.ds(r * 2 * NQ * 16 + NQ * 16, NQ * 16)], qb)
            for k in range(NQ):
                sl = pl.ds(k * 16, 16)
                acc2b[sl] = acc2b[sl] + qb[sl]
        lanes = lax.iota(jnp.int32, 16)
        row1 = jnp.zeros((16,), jnp.float32)
        row2 = jnp.zeros((16,), jnp.float32)
        for k in range(NQ):
            sl = pl.ds(k * 16, 16)
            row1 = jnp.where(lanes == k, jnp.sum(acc1b[sl]), row1)
            row2 = jnp.where(lanes == k, jnp.sum(acc2b[sl]), row2)
        nvec = jnp.full((16,), 0, jnp.int32) + nodesc
        row0 = jnp.zeros((16,), jnp.float32)
        for k in range(9):
            pltpu.sync_copy(qrefs[k].at[nvec], qb.at[pl.ds(k * 16, 16)])
            row0 = jnp.where(lanes == k, jnp.max(qb[pl.ds(k * 16, 16)]),
                             row0)
        st3[pl.ds(0, 16)] = row0
        st3[pl.ds(16, 16)] = row1
        st3[pl.ds(32, 16)] = row2
        pltpu.sync_copy(st3, out_hbm)


def _sc_stats(src, dst, cost, time, fast_pad, node_vec, rt_vec):
    mesh = plsc.VectorSubcoreMesh(core_axis_name="c", subcore_axis_name="s",
                                  num_cores=2, num_subcores=16)
    f32 = jnp.float32
    scratch = [
        pltpu.VMEM((EPT,), jnp.int32),
        pltpu.VMEM((EPT,), jnp.int32),
        pltpu.VMEM((EPT,), f32),
        pltpu.VMEM((EPT,), f32),
        pltpu.VMEM((NPAD,), f32),
        pltpu.VMEM((NPAD,), f32),
        pltpu.VMEM((NPAD,), f32),
        pltpu.VMEM((NPAD,), f32),
        pltpu.VMEM((NPAD,), f32),
        pltpu.VMEM((8, RPT), f32),
        pltpu.VMEM((RPT,), f32),
        pltpu.VMEM((RPT,), f32),
        pltpu.VMEM((RPT,), f32),
        pltpu.VMEM((NQ * 16,), f32),
        pltpu.VMEM((NQ * 16,), f32),
        pltpu.VMEM((NQ * 16,), f32),
        pltpu.VMEM((16,), jnp.int32),
        pltpu.VMEM((16,), f32),
        pltpu.VMEM((48,), f32),
        pltpu.VMEM_SHARED((NT, NPAD), f32),
        pltpu.VMEM_SHARED((NT, NPAD), f32),
    ] + [pltpu.VMEM_SHARED((NPAD,), f32) for _ in range(12)] + [
        pltpu.VMEM_SHARED((NT * 2 * NQ * 16,), f32),
    ]
    fn = pl.kernel(
        _sc_body,
        out_type=jax.ShapeDtypeStruct((48,), jnp.float32),
        mesh=mesh,
        scratch_types=scratch,
        compiler_params=pltpu.CompilerParams(needs_layout_passes=False),
    )
    return fn(src, dst, cost, time, fast_pad, node_vec, rt_vec)


def _finish_body(coef_ref, pT_ref, p2T_ref, pp0_ref, pp1_ref, out_ref):
    r2 = jnp.maximum(pT_ref[2:3, :], 0.0)
    r3 = jnp.maximum(pT_ref[3:4, :], 0.0)
    dn = (((1,), (1,)), ((), ()))
    v0 = lax.dot_general(r2, pp0_ref[:, :], dn,
                         preferred_element_type=jnp.float32)
    v1 = lax.dot_general(r3, pp1_ref[:, :], dn,
                         preferred_element_type=jnp.float32)
    bmat = jnp.concatenate([
        pT_ref[4:5, :], pT_ref[5:6, :], v0, v1,
        p2T_ref[0:1, :], p2T_ref[1:2, :], p2T_ref[2:3, :], p2T_ref[3:4, :],
    ], axis=0)
    out3 = lax.dot_general(coef_ref[:, :], bmat, (((1,), (0,)), ((), ())),
                           preferred_element_type=jnp.float32)
    p1 = pT_ref[1:2, :]
    out_ref[:, :] = out3[0:1] + p1 * out3[1:2] + (p1 * p1) * out3[2:3]


def _finish(coef, params_p, params_p2, params_pp):
    pT = params_p.T
    p2T = jnp.stack([params_p2[:, 0, 0], params_p2[:, 1, 0],
                     params_p2[:, 0, 1], params_p2[:, 1, 1]], axis=0)
    pp0 = params_pp[:, :, 0]
    pp1 = params_pp[:, :, 1]
    out = pl.pallas_call(
        _finish_body,
        out_shape=jax.ShapeDtypeStruct((1, DIM), jnp.float32),
    )(coef, pT, p2T, pp0, pp1)
    return out[0]


def kernel(edge_index, edge_cost, edge_time, fastest_time, params_p,
           params_p2, params_pp, node, dest, remaining_time):
    src = edge_index[0]
    dst = edge_index[1]
    rt0 = jnp.float32(remaining_time)
    fast_pad = jnp.pad(fastest_time, (0, NPAD - N))
    node_vec = jnp.full((16,), node, jnp.int32)
    rt_vec = jnp.full((16,), rt0, jnp.float32)

    res = _sc_stats(src, dst, edge_cost, edge_time, fast_pad, node_vec,
                    rt_vec).reshape(3, 16)
    row0, row1, row2 = res[0], res[1], res[2]
    dnode = row0[8]
    g0 = jnp.stack([row0[0], rt0, row0[2], row0[3], row0[4], row0[5],
                    row0[6], row0[7]])
    s1 = jnp.stack([row1[0], row1[9], row1[2], row1[3], row1[4], row1[5],
                    row1[6], row1[7]]) / dnode
    s2 = row2[:8] / dnode
    coef = jnp.stack([g0, s1, s2], axis=0)
    return _finish(coef, params_p, params_p2, params_pp)


# speedup vs baseline: 45.5241x; 1.1816x over previous
import jax
import jax.numpy as jnp
from jax import lax
from jax.experimental import pallas as pl
from jax.experimental.pallas import tpu as pltpu
from jax.experimental.pallas import tpu_sc as plsc

N = 10000
E = 160000
DIM = 256
NPAD = 10240
NT = 16
EPT = E // NT
RPT = NPAD // NT
NV = RPT // 16
NCHUNK = EPT // 16
BIG = 1e30
NQ = 10


def _fill(ref, nvec, val, dtype):
    vec = jnp.full((16,), val, dtype)

    def body(j, c):
        ref[pl.ds(j * 16, 16)] = vec
        return c

    lax.fori_loop(0, nvec, body, 0)


def _merge(mbref, rbase, bb, out_tmp, is_min):
    init = BIG if is_min else 0.0

    def _half(h, first):
        pltpu.sync_copy(mbref.at[pl.ds(h, 8), pl.ds(rbase, RPT)], bb)

        def body(v, c):
            acc = bb[0, pl.ds(v * 16, 16)]
            for r in range(1, 8):
                x = bb[r, pl.ds(v * 16, 16)]
                acc = jnp.minimum(acc, x) if is_min else acc + x
            if first:
                out_tmp[pl.ds(v * 16, 16)] = acc
            else:
                prev = out_tmp[pl.ds(v * 16, 16)]
                out_tmp[pl.ds(v * 16, 16)] = (
                    jnp.minimum(prev, acc) if is_min else prev + acc)
            return c

        lax.fori_loop(0, NV, body, 0)

    del init
    _half(0, True)
    _half(8, False)


def _sc_body(src_hbm, dst_hbm, cost_hbm, time_hbm, fast_hbm, node_hbm,
             rt_hbm, out_hbm,
             src_loc, dst_loc, cost_loc, time_loc,
             w0, w1, w2, w3, w4,
             bb, tmp1, tmp2, tmp3,
             qb, acc1b, acc2b, nv, rv, st3,
             mb0, mb1,
             q0, q1, q2, q3, q4, q5, q6, q7, q8, q9, q10, q11,
             accsh, zc, bc):
    sid = lax.axis_index("s")
    cid = lax.axis_index("c")
    ebase = sid * EPT
    rbase = sid * RPT

    pltpu.sync_copy(src_hbm.at[pl.ds(ebase, EPT)], src_loc)
    pltpu.sync_copy(dst_hbm.at[pl.ds(ebase, EPT)], dst_loc)
    pltpu.sync_copy(cost_hbm.at[pl.ds(ebase, EPT)], cost_loc)
    pltpu.sync_copy(time_hbm.at[pl.ds(ebase, EPT)], time_loc)
    pltpu.sync_copy(node_hbm, nv)
    pltpu.sync_copy(rt_hbm, rv)
    nodesc = nv[:][0]
    rt0 = rv[:][0]

    _fill(tmp3, NV, 0.0, jnp.float32)
    pltpu.sync_copy(tmp3, zc.at[pl.ds(rbase, RPT)])
    _fill(tmp3, NV, BIG, jnp.float32)
    pltpu.sync_copy(tmp3, bc.at[pl.ds(rbase, RPT)])
    plsc.subcore_barrier()

    pltpu.sync_copy(zc, w0)
    pltpu.sync_copy(zc, w1)
    pltpu.sync_copy(zc, w2)
    pltpu.sync_copy(bc, w3)
    pltpu.sync_copy(bc, w4)
    ones16 = jnp.ones((16,), jnp.float32)

    def p1_body(i, c):
        b = i * 16
        srcv = src_loc[pl.ds(b, 16)]
        costv = cost_loc[pl.ds(b, 16)]
        timev = time_loc[pl.ds(b, 16)]
        cnt, lastm = plsc.scan_count(srcv)
        ndup = plsc.all_reduce_population_count(lastm)[0]

        @pl.when(ndup == 16)
        def _():
            plsc.addupdate_scatter(w0, [srcv], ones16)
            plsc.addupdate_scatter(w1, [srcv], costv)
            plsc.addupdate_scatter(w2, [srcv], timev)
            curc = plsc.load_gather(w3, [srcv])
            plsc.store_scatter(w3, [srcv], jnp.minimum(curc, costv))
            curt = plsc.load_gather(w4, [srcv])
            plsc.store_scatter(w4, [srcv], jnp.minimum(curt, timev))

        @pl.when(ndup < 16)
        def _():
            cntf = cnt.astype(jnp.float32)
            mn = jnp.min(cntf).astype(jnp.int32)
            mx = jnp.max(cntf).astype(jnp.int32)

            def rbody(r, c2):
                rm = cnt == r
                plsc.addupdate_scatter(w0, [srcv], ones16, mask=rm)
                plsc.addupdate_scatter(w1, [srcv], costv, mask=rm)
                plsc.addupdate_scatter(w2, [srcv], timev, mask=rm)
                curc = plsc.load_gather(w3, [srcv])
                plsc.store_scatter(w3, [srcv], jnp.minimum(curc, costv),
                                   mask=rm)
                curt = plsc.load_gather(w4, [srcv])
                plsc.store_scatter(w4, [srcv], jnp.minimum(curt, timev),
                                   mask=rm)
                return c2

            lax.fori_loop(mn, mx + 1, rbody, 0)

        return c

    lax.fori_loop(0, NCHUNK, p1_body, 0)

    def ew(dst_ref, fn):
        def body(v, c):
            sl = pl.ds(v * 16, 16)
            dst_ref[sl] = fn(sl)
            return c

        lax.fori_loop(0, NV, body, 0)

    pltpu.sync_copy(w0, mb0.at[sid])
    pltpu.sync_copy(w1, mb1.at[sid])
    plsc.subcore_barrier()
    _merge(mb0, rbase, bb, tmp1, False)
    ew(tmp2, lambda sl: jnp.maximum(tmp1[sl], 1.0))
    pltpu.sync_copy(tmp1, q11.at[pl.ds(rbase, RPT)])
    pltpu.sync_copy(tmp2, q8.at[pl.ds(rbase, RPT)])
    _merge(mb1, rbase, bb, tmp3, False)
    ew(tmp3, lambda sl: tmp3[sl] / tmp2[sl])
    pltpu.sync_copy(tmp3, q2.at[pl.ds(rbase, RPT)])
    plsc.subcore_barrier()

    pltpu.sync_copy(w2, mb0.at[sid])
    pltpu.sync_copy(w3, mb1.at[sid])
    plsc.subcore_barrier()
    _merge(mb0, rbase, bb, tmp3, False)
    ew(tmp3, lambda sl: tmp3[sl] / tmp2[sl])
    pltpu.sync_copy(tmp3, q3.at[pl.ds(rbase, RPT)])
    _merge(mb1, rbase, bb, tmp3, True)
    ew(tmp3, lambda sl: jnp.where(tmp1[sl] > 0.0, tmp3[sl], 10000.0))
    pltpu.sync_copy(tmp3, q4.at[pl.ds(rbase, RPT)])
    plsc.subcore_barrier()

    pltpu.sync_copy(w4, mb0.at[sid])
    plsc.subcore_barrier()
    _merge(mb0, rbase, bb, tmp3, True)
    ew(tmp3, lambda sl: jnp.where(tmp1[sl] > 0.0, tmp3[sl], 10000.0))
    pltpu.sync_copy(tmp3, q7.at[pl.ds(rbase, RPT)])
    pltpu.sync_copy(fast_hbm.at[pl.ds(rbase, RPT)], tmp3)
    pltpu.sync_copy(tmp3, q0.at[pl.ds(rbase, RPT)])
    plsc.subcore_barrier()

    pltpu.sync_copy(q4, w0)
    pltpu.sync_copy(q7, w1)
    pltpu.sync_copy(bc, w2)
    pltpu.sync_copy(bc, w3)

    def p2_body(i, c):
        b = i * 16
        srcv = src_loc[pl.ds(b, 16)]
        costv = cost_loc[pl.ds(b, 16)]
        timev = time_loc[pl.ds(b, 16)]
        mc = plsc.load_gather(w0, [srcv])
        mt = plsc.load_gather(w1, [srcv])
        candt = jnp.where(costv <= mc, timev, BIG)
        candc = jnp.where(timev <= mt, costv, BIG)
        cnt, lastm = plsc.scan_count(srcv)
        ndup = plsc.all_reduce_population_count(lastm)[0]

        @pl.when(ndup == 16)
        def _():
            cur1 = plsc.load_gather(w2, [srcv])
            plsc.store_scatter(w2, [srcv], jnp.minimum(cur1, candt))
            cur2 = plsc.load_gather(w3, [srcv])
            plsc.store_scatter(w3, [srcv], jnp.minimum(cur2, candc))

        @pl.when(ndup < 16)
        def _():
            cntf = cnt.astype(jnp.float32)
            mn = jnp.min(cntf).astype(jnp.int32)
            mx = jnp.max(cntf).astype(jnp.int32)

            def rbody(r, c2):
                rm = cnt == r
                cur1 = plsc.load_gather(w2, [srcv])
                plsc.store_scatter(w2, [srcv], jnp.minimum(cur1, candt),
                                   mask=rm)
                cur2 = plsc.load_gather(w3, [srcv])
                plsc.store_scatter(w3, [srcv], jnp.minimum(cur2, candc),
                                   mask=rm)
                return c2

            lax.fori_loop(mn, mx + 1, rbody, 0)

        return c

    lax.fori_loop(0, NCHUNK, p2_body, 0)
    pltpu.sync_copy(w2, mb0.at[sid])
    pltpu.sync_copy(w3, mb1.at[sid])
    plsc.subcore_barrier()
    pltpu.sync_copy(q11.at[pl.ds(rbase, RPT)], tmp1)
    _merge(mb0, rbase, bb, tmp3, True)
    ew(tmp3, lambda sl: jnp.where(tmp1[sl] > 0.0, tmp3[sl], 0.0))
    pltpu.sync_copy(tmp3, q5.at[pl.ds(rbase, RPT)])
    _merge(mb1, rbase, bb, tmp3, True)
    ew(tmp3, lambda sl: jnp.where(tmp1[sl] > 0.0, tmp3[sl], 0.0))
    pltpu.sync_copy(tmp3, q6.at[pl.ds(rbase, RPT)])
    plsc.subcore_barrier()

    pltpu.sync_copy(zc, w0)
    pltpu.sync_copy(zc, w1)

    def p3_body(i, c):
        b = i * 16
        srcv = src_loc[pl.ds(b, 16)]
        m = srcv == nodesc
        anyv = plsc.all_reduce_population_count(m)[0]

        @pl.when(anyv > 0)
        def _():
            dstv = dst_loc[pl.ds(b, 16)]
            timev = time_loc[pl.ds(b, 16)]
            mf = jnp.where(m, 1.0, 0.0)
            cnt, _ = plsc.scan_count(dstv)
            cntf = cnt.astype(jnp.float32)
            mn = jnp.min(cntf).astype(jnp.int32)
            mx = jnp.max(cntf).astype(jnp.int32)

            def rbody(r, c2):
                rm = cnt == r
                plsc.addupdate_scatter(w0, [dstv], mf, mask=rm)
                plsc.addupdate_scatter(w1, [dstv], mf * (rt0 - timev),
                                       mask=rm)
                return c2

            lax.fori_loop(mn, mx + 1, rbody, 0)

        return c

    lax.fori_loop(0, NCHUNK, p3_body, 0)
    pltpu.sync_copy(w0, mb0.at[sid])
    pltpu.sync_copy(w1, mb1.at[sid])
    plsc.subcore_barrier()
    pltpu.sync_copy(q8.at[pl.ds(rbase, RPT)], tmp2)
    _merge(mb0, rbase, bb, tmp1, False)
    ew(tmp3, lambda sl: tmp1[sl] / tmp2[sl])
    pltpu.sync_copy(tmp3, q10.at[pl.ds(rbase, RPT)])
    _merge(mb1, rbase, bb, tmp3, False)
    ew(tmp3, lambda sl: tmp3[sl] / jnp.maximum(tmp1[sl], 1.0))
    pltpu.sync_copy(tmp3, q9.at[pl.ds(rbase, RPT)])
    plsc.subcore_barrier()

    pltpu.sync_copy(q10, w2)
    pltpu.sync_copy(q9, w3)
    pltpu.sync_copy(zc, w0)
    pltpu.sync_copy(zc, w1)

    def p4_body(i, c):
        b = i * 16
        srcv = src_loc[pl.ds(b, 16)]
        wv = plsc.load_gather(w2, [srcv])
        anyv = plsc.all_reduce_population_count(wv > 0.0)[0]

        @pl.when(anyv > 0)
        def _():
            dstv = dst_loc[pl.ds(b, 16)]
            timev = time_loc[pl.ds(b, 16)]
            gf = jnp.where(wv > 0.0, 1.0, 0.0)
            t1 = plsc.load_gather(w3, [srcv])
            cnt, _ = plsc.scan_count(dstv)
            cntf = cnt.astype(jnp.float32)
            mn = jnp.min(cntf).astype(jnp.int32)
            mx = jnp.max(cntf).astype(jnp.int32)

            def rbody(r, c2):
                rm = cnt == r
                plsc.addupdate_scatter(w0, [dstv], gf, mask=rm)
                plsc.addupdate_scatter(w1, [dstv], gf * (t1 - timev),
                                       mask=rm)
                return c2

            lax.fori_loop(mn, mx + 1, rbody, 0)

        return c

    lax.fori_loop(0, NCHUNK, p4_body, 0)
    pltpu.sync_copy(w0, mb0.at[sid])
    pltpu.sync_copy(w1, mb1.at[sid])
    plsc.subcore_barrier()
    _merge(mb0, rbase, bb, tmp1, False)
    _merge(mb1, rbase, bb, tmp3, False)
    ew(tmp3, lambda sl: tmp3[sl] / jnp.maximum(tmp1[sl], 1.0))
    pltpu.sync_copy(tmp3, q1.at[pl.ds(rbase, RPT)])
    plsc.subcore_barrier()

    z16 = jnp.zeros((16,), jnp.float32)
    for k in range(NQ):
        acc1b[pl.ds(k * 16, 16)] = z16
        acc2b[pl.ds(k * 16, 16)] = z16
    qrefs = (q0, q1, q2, q3, q4, q5, q6, q7, q8, q9)

    def p5_body(i, c):
        b = i * 16
        srcv = src_loc[pl.ds(b, 16)]
        wv = plsc.load_gather(w2, [srcv])
        mbm = srcv == nodesc
        mbf = jnp.where(mbm, 1.0, 0.0)
        act = plsc.all_reduce_population_count(
            jnp.logical_or(wv > 0.0, mbm))[0]

        @pl.when(act > 0)
        def _():
            dstv = dst_loc[pl.ds(b, 16)]
            for k in range(NQ):
                pltpu.sync_copy(qrefs[k].at[dstv],
                                qb.at[pl.ds(k * 16, 16)])
            for k in range(NQ):
                sl = pl.ds(k * 16, 16)
                qv = qb[sl]
                acc2b[sl] = acc2b[sl] + wv * qv
                acc1b[sl] = acc1b[sl] + mbf * qv

        return c

    lax.fori_loop(0, NCHUNK, p5_body, 0)
    pltpu.sync_copy(acc1b, accsh.at[pl.ds(sid * 2 * NQ * 16, NQ * 16)])
    pltpu.sync_copy(acc2b,
                    accsh.at[pl.ds(sid * 2 * NQ * 16 + NQ * 16, NQ * 16)])
    plsc.subcore_barrier()

    @pl.when(jnp.logical_and(sid == 0, cid == 0))
    def _():
        for k in range(NQ):
            acc1b[pl.ds(k * 16, 16)] = z16
            acc2b[pl.ds(k * 16, 16)] = z16
        for r in range(NT):
            pltpu.sync_copy(accsh.at[pl.ds(r * 2 * NQ * 16, NQ * 16)], qb)
            for k in range(NQ):
                sl = pl.ds(k * 16, 16)
                acc1b[sl] = acc1b[sl] + qb[sl]
            pltpu.sync_copy(
                accsh.at[pl.ds(r * 2 * NQ * 16 + NQ * 16, NQ * 16)], qb)
            for k in range(NQ):
                sl = pl.ds(k * 16, 16)
                acc2b[sl] = acc2b[sl] + qb[sl]
        lanes = lax.iota(jnp.int32, 16)
        row1 = jnp.zeros((16,), jnp.float32)
        row2 = jnp.zeros((16,), jnp.float32)
        for k in range(NQ):
            sl = pl.ds(k * 16, 16)
            row1 = jnp.where(lanes == k, jnp.sum(acc1b[sl]), row1)
            row2 = jnp.where(lanes == k, jnp.sum(acc2b[sl]), row2)
        nvec = jnp.full((16,), 0, jnp.int32) + nodesc
        row0 = jnp.zeros((16,), jnp.float32)
        for k in range(9):
            pltpu.sync_copy(qrefs[k].at[nvec], qb.at[pl.ds(k * 16, 16)])
            row0 = jnp.where(lanes == k, jnp.max(qb[pl.ds(k * 16, 16)]),
                             row0)
        st3[pl.ds(0, 16)] = row0
        st3[pl.ds(16, 16)] = row1
        st3[pl.ds(32, 16)] = row2
        pltpu.sync_copy(st3, out_hbm)


def _sc_stats(src, dst, cost, time, fast_pad, node_vec, rt_vec):
    mesh = plsc.VectorSubcoreMesh(core_axis_name="c", subcore_axis_name="s",
                                  num_cores=2, num_subcores=16)
    f32 = jnp.float32
    scratch = [
        pltpu.VMEM((EPT,), jnp.int32),
        pltpu.VMEM((EPT,), jnp.int32),
        pltpu.VMEM((EPT,), f32),
        pltpu.VMEM((EPT,), f32),
        pltpu.VMEM((NPAD,), f32),
        pltpu.VMEM((NPAD,), f32),
        pltpu.VMEM((NPAD,), f32),
        pltpu.VMEM((NPAD,), f32),
        pltpu.VMEM((NPAD,), f32),
        pltpu.VMEM((8, RPT), f32),
        pltpu.VMEM((RPT,), f32),
        pltpu.VMEM((RPT,), f32),
        pltpu.VMEM((RPT,), f32),
        pltpu.VMEM((NQ * 16,), f32),
        pltpu.VMEM((NQ * 16,), f32),
        pltpu.VMEM((NQ * 16,), f32),
        pltpu.VMEM((16,), jnp.int32),
        pltpu.VMEM((16,), f32),
        pltpu.VMEM((48,), f32),
        pltpu.VMEM_SHARED((NT, NPAD), f32),
        pltpu.VMEM_SHARED((NT, NPAD), f32),
    ] + [pltpu.VMEM_SHARED((NPAD,), f32) for _ in range(12)] + [
        pltpu.VMEM_SHARED((NT * 2 * NQ * 16,), f32),
        pltpu.VMEM_SHARED((NPAD,), f32),
        pltpu.VMEM_SHARED((NPAD,), f32),
    ]
    fn = pl.kernel(
        _sc_body,
        out_type=jax.ShapeDtypeStruct((48,), jnp.float32),
        mesh=mesh,
        scratch_types=scratch,
        compiler_params=pltpu.CompilerParams(needs_layout_passes=False),
    )
    return fn(src, dst, cost, time, fast_pad, node_vec, rt_vec)


def _finish_body(coef_ref, pT_ref, p2T_ref, pp0_ref, pp1_ref, out_ref):
    r2 = jnp.maximum(pT_ref[2:3, :], 0.0)
    r3 = jnp.maximum(pT_ref[3:4, :], 0.0)
    dn = (((1,), (1,)), ((), ()))
    v0 = lax.dot_general(r2, pp0_ref[:, :], dn,
                         preferred_element_type=jnp.float32)
    v1 = lax.dot_general(r3, pp1_ref[:, :], dn,
                         preferred_element_type=jnp.float32)
    bmat = jnp.concatenate([
        pT_ref[4:5, :], pT_ref[5:6, :], v0, v1,
        p2T_ref[0:1, :], p2T_ref[1:2, :], p2T_ref[2:3, :], p2T_ref[3:4, :],
    ], axis=0)
    out3 = lax.dot_general(coef_ref[:, :], bmat, (((1,), (0,)), ((), ())),
                           preferred_element_type=jnp.float32)
    p1 = pT_ref[1:2, :]
    out_ref[:, :] = out3[0:1] + p1 * out3[1:2] + (p1 * p1) * out3[2:3]


def _finish(coef, params_p, params_p2, params_pp):
    pT = params_p.T
    p2T = jnp.stack([params_p2[:, 0, 0], params_p2[:, 1, 0],
                     params_p2[:, 0, 1], params_p2[:, 1, 1]], axis=0)
    pp0 = params_pp[:, :, 0]
    pp1 = params_pp[:, :, 1]
    out = pl.pallas_call(
        _finish_body,
        out_shape=jax.ShapeDtypeStruct((1, DIM), jnp.float32),
    )(coef, pT, p2T, pp0, pp1)
    return out[0]


def kernel(edge_index, edge_cost, edge_time, fastest_time, params_p,
           params_p2, params_pp, node, dest, remaining_time):
    src = edge_index[0]
    dst = edge_index[1]
    rt0 = jnp.float32(remaining_time)
    fast_pad = jnp.pad(fastest_time, (0, NPAD - N))
    node_vec = jnp.full((16,), node, jnp.int32)
    rt_vec = jnp.full((16,), rt0, jnp.float32)

    res = _sc_stats(src, dst, edge_cost, edge_time, fast_pad, node_vec,
                    rt_vec).reshape(3, 16)
    row0, row1, row2 = res[0], res[1], res[2]
    dnode = row0[8]
    g0 = jnp.stack([row0[0], rt0, row0[2], row0[3], row0[4], row0[5],
                    row0[6], row0[7]])
    s1 = jnp.stack([row1[0], row1[9], row1[2], row1[3], row1[4], row1[5],
                    row1[6], row1[7]]) / dnode
    s2 = row2[:8] / dnode
    coef = jnp.stack([g0, s1, s2], axis=0)
    return _finish(coef, params_p, params_p2, params_pp)


# speedup vs baseline: 61.4458x; 1.3497x over previous
import jax
import jax.numpy as jnp
from jax import lax
from jax.experimental import pallas as pl
from jax.experimental.pallas import tpu as pltpu
from jax.experimental.pallas import tpu_sc as plsc

N = 10000
E = 160000
DIM = 256
NPAD = 10240
NT = 16
EPT = E // NT
RPT = NPAD // NT
NV = RPT // 16
NCHUNK = EPT // 16
BIG = 1e30
NQ = 10


def _fill(ref, nvec, val, dtype):
    vec = jnp.full((16,), val, dtype)

    def body(j, c):
        ref[pl.ds(j * 16, 16)] = vec
        return c

    lax.fori_loop(0, nvec, body, 0)


def _merge(mbref, rbase, bb, out_tmp, is_min):
    init = BIG if is_min else 0.0

    def _half(h, first):
        pltpu.sync_copy(mbref.at[pl.ds(h, 8), pl.ds(rbase, RPT)], bb)

        def body(v, c):
            acc = bb[0, pl.ds(v * 16, 16)]
            for r in range(1, 8):
                x = bb[r, pl.ds(v * 16, 16)]
                acc = jnp.minimum(acc, x) if is_min else acc + x
            if first:
                out_tmp[pl.ds(v * 16, 16)] = acc
            else:
                prev = out_tmp[pl.ds(v * 16, 16)]
                out_tmp[pl.ds(v * 16, 16)] = (
                    jnp.minimum(prev, acc) if is_min else prev + acc)
            return c

        lax.fori_loop(0, NV, body, 0)

    del init
    _half(0, True)
    _half(8, False)


def _sc_body(src_hbm, dst_hbm, cost_hbm, time_hbm, fast_hbm, node_hbm,
             rt_hbm, out_hbm,
             src_loc, dst_loc, cost_loc, time_loc,
             w0, w1, w2, w3, w4,
             bb, tmp1, tmp2, tmp3,
             qb, acc1b, acc2b, nv, rv, st3,
             mb0, mb1,
             q0, q1, q2, q3, q4, q5, q6, q7, q8, q9, q10, q11,
             accsh, zc, bc):
    sid = lax.axis_index("s")
    cid = lax.axis_index("c")
    ebase = sid * EPT
    rbase = sid * RPT

    pltpu.sync_copy(src_hbm.at[pl.ds(ebase, EPT)], src_loc)
    pltpu.sync_copy(dst_hbm.at[pl.ds(ebase, EPT)], dst_loc)
    pltpu.sync_copy(cost_hbm.at[pl.ds(ebase, EPT)], cost_loc)
    pltpu.sync_copy(time_hbm.at[pl.ds(ebase, EPT)], time_loc)
    pltpu.sync_copy(node_hbm, nv)
    pltpu.sync_copy(rt_hbm, rv)
    nodesc = nv[:][0]
    rt0 = rv[:][0]

    _fill(tmp3, NV, 0.0, jnp.float32)
    pltpu.sync_copy(tmp3, zc.at[pl.ds(rbase, RPT)])
    _fill(tmp3, NV, BIG, jnp.float32)
    pltpu.sync_copy(tmp3, bc.at[pl.ds(rbase, RPT)])
    plsc.subcore_barrier()

    pltpu.sync_copy(zc, w0)
    pltpu.sync_copy(zc, w1)
    pltpu.sync_copy(zc, w2)
    pltpu.sync_copy(bc, w3)
    pltpu.sync_copy(bc, w4)
    ones16 = jnp.ones((16,), jnp.float32)

    def p1_body(i, c):
        b = i * 16
        srcv = src_loc[pl.ds(b, 16)]
        costv = cost_loc[pl.ds(b, 16)]
        timev = time_loc[pl.ds(b, 16)]
        cnt, lastm = plsc.scan_count(srcv)
        ndup = plsc.all_reduce_population_count(lastm)[0]

        @pl.when(ndup == 16)
        def _():
            plsc.addupdate_scatter(w0, [srcv], ones16)
            plsc.addupdate_scatter(w1, [srcv], costv)
            plsc.addupdate_scatter(w2, [srcv], timev)
            curc = plsc.load_gather(w3, [srcv])
            plsc.store_scatter(w3, [srcv], jnp.minimum(curc, costv))
            curt = plsc.load_gather(w4, [srcv])
            plsc.store_scatter(w4, [srcv], jnp.minimum(curt, timev))

        @pl.when(ndup < 16)
        def _():
            cntf = cnt.astype(jnp.float32)
            mn = jnp.min(cntf).astype(jnp.int32)
            mx = jnp.max(cntf).astype(jnp.int32)

            def rbody(r, c2):
                rm = cnt == r
                plsc.addupdate_scatter(w0, [srcv], ones16, mask=rm)
                plsc.addupdate_scatter(w1, [srcv], costv, mask=rm)
                plsc.addupdate_scatter(w2, [srcv], timev, mask=rm)
                curc = plsc.load_gather(w3, [srcv])
                plsc.store_scatter(w3, [srcv], jnp.minimum(curc, costv),
                                   mask=rm)
                curt = plsc.load_gather(w4, [srcv])
                plsc.store_scatter(w4, [srcv], jnp.minimum(curt, timev),
                                   mask=rm)
                return c2

            lax.fori_loop(mn, mx + 1, rbody, 0)

        return c

    lax.fori_loop(0, NCHUNK, p1_body, 0)

    def ew(dst_ref, fn):
        def body(v, c):
            sl = pl.ds(v * 16, 16)
            dst_ref[sl] = fn(sl)
            return c

        lax.fori_loop(0, NV, body, 0)

    pltpu.sync_copy(w0, mb0.at[sid])
    pltpu.sync_copy(w1, mb1.at[sid])
    plsc.subcore_barrier()
    _merge(mb0, rbase, bb, tmp1, False)
    ew(tmp2, lambda sl: jnp.maximum(tmp1[sl], 1.0))
    pltpu.sync_copy(tmp1, q11.at[pl.ds(rbase, RPT)])
    pltpu.sync_copy(tmp2, q8.at[pl.ds(rbase, RPT)])
    _merge(mb1, rbase, bb, tmp3, False)
    ew(tmp3, lambda sl: tmp3[sl] / tmp2[sl])
    pltpu.sync_copy(tmp3, q2.at[pl.ds(rbase, RPT)])
    plsc.subcore_barrier()

    pltpu.sync_copy(w2, mb0.at[sid])
    pltpu.sync_copy(w3, mb1.at[sid])
    plsc.subcore_barrier()
    _merge(mb0, rbase, bb, tmp3, False)
    ew(tmp3, lambda sl: tmp3[sl] / tmp2[sl])
    pltpu.sync_copy(tmp3, q3.at[pl.ds(rbase, RPT)])
    _merge(mb1, rbase, bb, tmp3, True)
    ew(tmp3, lambda sl: jnp.where(tmp1[sl] > 0.0, tmp3[sl], 10000.0))
    pltpu.sync_copy(tmp3, q4.at[pl.ds(rbase, RPT)])
    plsc.subcore_barrier()

    pltpu.sync_copy(w4, mb0.at[sid])
    plsc.subcore_barrier()
    _merge(mb0, rbase, bb, tmp3, True)
    ew(tmp3, lambda sl: jnp.where(tmp1[sl] > 0.0, tmp3[sl], 10000.0))
    pltpu.sync_copy(tmp3, q7.at[pl.ds(rbase, RPT)])
    pltpu.sync_copy(fast_hbm.at[pl.ds(rbase, RPT)], tmp3)
    pltpu.sync_copy(tmp3, q0.at[pl.ds(rbase, RPT)])
    plsc.subcore_barrier()

    pltpu.sync_copy(q4, w0)
    pltpu.sync_copy(q7, w1)
    pltpu.sync_copy(bc, w2)
    pltpu.sync_copy(bc, w3)

    def p2_body(i, c):
        b = i * 16
        srcv = src_loc[pl.ds(b, 16)]
        costv = cost_loc[pl.ds(b, 16)]
        timev = time_loc[pl.ds(b, 16)]
        mc = plsc.load_gather(w0, [srcv])
        mt = plsc.load_gather(w1, [srcv])
        candt = jnp.where(costv <= mc, timev, BIG)
        candc = jnp.where(timev <= mt, costv, BIG)
        cnt, lastm = plsc.scan_count(srcv)
        ndup = plsc.all_reduce_population_count(lastm)[0]

        @pl.when(ndup == 16)
        def _():
            cur1 = plsc.load_gather(w2, [srcv])
            plsc.store_scatter(w2, [srcv], jnp.minimum(cur1, candt))
            cur2 = plsc.load_gather(w3, [srcv])
            plsc.store_scatter(w3, [srcv], jnp.minimum(cur2, candc))

        @pl.when(ndup < 16)
        def _():
            cntf = cnt.astype(jnp.float32)
            mn = jnp.min(cntf).astype(jnp.int32)
            mx = jnp.max(cntf).astype(jnp.int32)

            def rbody(r, c2):
                rm = cnt == r
                cur1 = plsc.load_gather(w2, [srcv])
                plsc.store_scatter(w2, [srcv], jnp.minimum(cur1, candt),
                                   mask=rm)
                cur2 = plsc.load_gather(w3, [srcv])
                plsc.store_scatter(w3, [srcv], jnp.minimum(cur2, candc),
                                   mask=rm)
                return c2

            lax.fori_loop(mn, mx + 1, rbody, 0)

        return c

    lax.fori_loop(0, NCHUNK, p2_body, 0)
    pltpu.sync_copy(w2, mb0.at[sid])
    pltpu.sync_copy(w3, mb1.at[sid])
    plsc.subcore_barrier()
    pltpu.sync_copy(q11.at[pl.ds(rbase, RPT)], tmp1)
    _merge(mb0, rbase, bb, tmp3, True)
    ew(tmp3, lambda sl: jnp.where(tmp1[sl] > 0.0, tmp3[sl], 0.0))
    pltpu.sync_copy(tmp3, q5.at[pl.ds(rbase, RPT)])
    _merge(mb1, rbase, bb, tmp3, True)
    ew(tmp3, lambda sl: jnp.where(tmp1[sl] > 0.0, tmp3[sl], 0.0))
    pltpu.sync_copy(tmp3, q6.at[pl.ds(rbase, RPT)])
    plsc.subcore_barrier()

    pltpu.sync_copy(zc, w0)
    pltpu.sync_copy(zc, w1)

    def p3_chunk(b, m):
        dstv = dst_loc[pl.ds(b, 16)]
        timev = time_loc[pl.ds(b, 16)]
        mf = jnp.where(m, 1.0, 0.0)
        cnt, _ = plsc.scan_count(dstv)
        cntf = cnt.astype(jnp.float32)
        mn = jnp.min(cntf).astype(jnp.int32)
        mx = jnp.max(cntf).astype(jnp.int32)

        def rbody(r, c2):
            rm = cnt == r
            plsc.addupdate_scatter(w0, [dstv], mf, mask=rm)
            plsc.addupdate_scatter(w1, [dstv], mf * (rt0 - timev),
                                   mask=rm)
            return c2

        lax.fori_loop(mn, mx + 1, rbody, 0)

    def p3_body(i, c):
        b = i * 80
        ms = [src_loc[pl.ds(b + 16 * j, 16)] == nodesc for j in range(5)]
        mall = ms[0]
        for j in range(1, 5):
            mall = jnp.logical_or(mall, ms[j])
        anyv = plsc.all_reduce_population_count(mall)[0]

        @pl.when(anyv > 0)
        def _():
            for j in range(5):
                anyj = plsc.all_reduce_population_count(ms[j])[0]

                @pl.when(anyj > 0)
                def _(bj=b + 16 * j, mj=ms[j]):
                    p3_chunk(bj, mj)

        return c

    lax.fori_loop(0, NCHUNK // 5, p3_body, 0)
    pltpu.sync_copy(w0, mb0.at[sid])
    pltpu.sync_copy(w1, mb1.at[sid])
    plsc.subcore_barrier()
    pltpu.sync_copy(q8.at[pl.ds(rbase, RPT)], tmp2)
    _merge(mb0, rbase, bb, tmp1, False)
    ew(tmp3, lambda sl: tmp1[sl] / tmp2[sl])
    pltpu.sync_copy(tmp3, q10.at[pl.ds(rbase, RPT)])
    _merge(mb1, rbase, bb, tmp3, False)
    ew(tmp3, lambda sl: tmp3[sl] / jnp.maximum(tmp1[sl], 1.0))
    pltpu.sync_copy(tmp3, q9.at[pl.ds(rbase, RPT)])
    plsc.subcore_barrier()

    pltpu.sync_copy(q10, w2)
    pltpu.sync_copy(q9, w3)
    pltpu.sync_copy(zc, w0)
    pltpu.sync_copy(zc, w1)

    def p4_chunk(b, wv):
        srcv = src_loc[pl.ds(b, 16)]
        dstv = dst_loc[pl.ds(b, 16)]
        timev = time_loc[pl.ds(b, 16)]
        gf = jnp.where(wv > 0.0, 1.0, 0.0)
        t1 = plsc.load_gather(w3, [srcv])
        cnt, _ = plsc.scan_count(dstv)
        cntf = cnt.astype(jnp.float32)
        mn = jnp.min(cntf).astype(jnp.int32)
        mx = jnp.max(cntf).astype(jnp.int32)

        def rbody(r, c2):
            rm = cnt == r
            plsc.addupdate_scatter(w0, [dstv], gf, mask=rm)
            plsc.addupdate_scatter(w1, [dstv], gf * (t1 - timev),
                                   mask=rm)
            return c2

        lax.fori_loop(mn, mx + 1, rbody, 0)

    def p4_body(i, c):
        b = i * 80
        wvs = [plsc.load_gather(w2, [src_loc[pl.ds(b + 16 * j, 16)]])
               for j in range(5)]
        gs = [wv > 0.0 for wv in wvs]
        gall = gs[0]
        for j in range(1, 5):
            gall = jnp.logical_or(gall, gs[j])
        anyv = plsc.all_reduce_population_count(gall)[0]

        @pl.when(anyv > 0)
        def _():
            for j in range(5):
                anyj = plsc.all_reduce_population_count(gs[j])[0]

                @pl.when(anyj > 0)
                def _(bj=b + 16 * j, wvj=wvs[j]):
                    p4_chunk(bj, wvj)

        return c

    lax.fori_loop(0, NCHUNK // 5, p4_body, 0)
    pltpu.sync_copy(w0, mb0.at[sid])
    pltpu.sync_copy(w1, mb1.at[sid])
    plsc.subcore_barrier()
    _merge(mb0, rbase, bb, tmp1, False)
    _merge(mb1, rbase, bb, tmp3, False)
    ew(tmp3, lambda sl: tmp3[sl] / jnp.maximum(tmp1[sl], 1.0))
    pltpu.sync_copy(tmp3, q1.at[pl.ds(rbase, RPT)])
    plsc.subcore_barrier()

    z16 = jnp.zeros((16,), jnp.float32)
    for k in range(NQ):
        acc1b[pl.ds(k * 16, 16)] = z16
        acc2b[pl.ds(k * 16, 16)] = z16
    qrefs = (q0, q1, q2, q3, q4, q5, q6, q7, q8, q9)

    def p5_chunk(b, wv, mbm):
        mbf = jnp.where(mbm, 1.0, 0.0)
        dstv = dst_loc[pl.ds(b, 16)]
        for k in range(NQ):
            pltpu.sync_copy(qrefs[k].at[dstv], qb.at[pl.ds(k * 16, 16)])
        for k in range(NQ):
            sl = pl.ds(k * 16, 16)
            qv = qb[sl]
            acc2b[sl] = acc2b[sl] + wv * qv
            acc1b[sl] = acc1b[sl] + mbf * qv

    def p5_body(i, c):
        b = i * 80
        srcs = [src_loc[pl.ds(b + 16 * j, 16)] for j in range(5)]
        wvs = [plsc.load_gather(w2, [sv]) for sv in srcs]
        mbs = [sv == nodesc for sv in srcs]
        acts = [jnp.logical_or(wvs[j] > 0.0, mbs[j]) for j in range(5)]
        aall = acts[0]
        for j in range(1, 5):
            aall = jnp.logical_or(aall, acts[j])
        anyv = plsc.all_reduce_population_count(aall)[0]

        @pl.when(anyv > 0)
        def _():
            for j in range(5):
                anyj = plsc.all_reduce_population_count(acts[j])[0]

                @pl.when(anyj > 0)
                def _(bj=b + 16 * j, wvj=wvs[j], mbj=mbs[j]):
                    p5_chunk(bj, wvj, mbj)

        return c

    lax.fori_loop(0, NCHUNK // 5, p5_body, 0)
    pltpu.sync_copy(acc1b, accsh.at[pl.ds(sid * 2 * NQ * 16, NQ * 16)])
    pltpu.sync_copy(acc2b,
                    accsh.at[pl.ds(sid * 2 * NQ * 16 + NQ * 16, NQ * 16)])
    plsc.subcore_barrier()

    @pl.when(jnp.logical_and(sid == 0, cid == 0))
    def _():
        for k in range(NQ):
            acc1b[pl.ds(k * 16, 16)] = z16
            acc2b[pl.ds(k * 16, 16)] = z16
        for r in range(NT):
            pltpu.sync_copy(accsh.at[pl.ds(r * 2 * NQ * 16, NQ * 16)], qb)
            for k in range(NQ):
                sl = pl.ds(k * 16, 16)
                acc1b[sl] = acc1b[sl] + qb[sl]
            pltpu.sync_copy(
                accsh.at[pl.ds(r * 2 * NQ * 16 + NQ * 16, NQ * 16)], qb)
            for k in range(NQ):
                sl = pl.ds(k * 16, 16)
                acc2b[sl] = acc2b[sl] + qb[sl]
        lanes = lax.iota(jnp.int32, 16)
        row1 = jnp.zeros((16,), jnp.float32)
        row2 = jnp.zeros((16,), jnp.float32)
        for k in range(NQ):
            sl = pl.ds(k * 16, 16)
            row1 = jnp.where(lanes == k, jnp.sum(acc1b[sl]), row1)
            row2 = jnp.where(lanes == k, jnp.sum(acc2b[sl]), row2)
        nvec = jnp.full((16,), 0, jnp.int32) + nodesc
        row0 = jnp.zeros((16,), jnp.float32)
        for k in range(9):
            pltpu.sync_copy(qrefs[k].at[nvec], qb.at[pl.ds(k * 16, 16)])
            row0 = jnp.where(lanes == k, jnp.max(qb[pl.ds(k * 16, 16)]),
                             row0)
        st3[pl.ds(0, 16)] = row0
        st3[pl.ds(16, 16)] = row1
        st3[pl.ds(32, 16)] = row2
        pltpu.sync_copy(st3, out_hbm)


def _sc_stats(src, dst, cost, time, fast_pad, node_vec, rt_vec):
    mesh = plsc.VectorSubcoreMesh(core_axis_name="c", subcore_axis_name="s",
                                  num_cores=2, num_subcores=16)
    f32 = jnp.float32
    scratch = [
        pltpu.VMEM((EPT,), jnp.int32),
        pltpu.VMEM((EPT,), jnp.int32),
        pltpu.VMEM((EPT,), f32),
        pltpu.VMEM((EPT,), f32),
        pltpu.VMEM((NPAD,), f32),
        pltpu.VMEM((NPAD,), f32),
        pltpu.VMEM((NPAD,), f32),
        pltpu.VMEM((NPAD,), f32),
        pltpu.VMEM((NPAD,), f32),
        pltpu.VMEM((8, RPT), f32),
        pltpu.VMEM((RPT,), f32),
        pltpu.VMEM((RPT,), f32),
        pltpu.VMEM((RPT,), f32),
        pltpu.VMEM((NQ * 16,), f32),
        pltpu.VMEM((NQ * 16,), f32),
        pltpu.VMEM((NQ * 16,), f32),
        pltpu.VMEM((16,), jnp.int32),
        pltpu.VMEM((16,), f32),
        pltpu.VMEM((48,), f32),
        pltpu.VMEM_SHARED((NT, NPAD), f32),
        pltpu.VMEM_SHARED((NT, NPAD), f32),
    ] + [pltpu.VMEM_SHARED((NPAD,), f32) for _ in range(12)] + [
        pltpu.VMEM_SHARED((NT * 2 * NQ * 16,), f32),
        pltpu.VMEM_SHARED((NPAD,), f32),
        pltpu.VMEM_SHARED((NPAD,), f32),
    ]
    fn = pl.kernel(
        _sc_body,
        out_type=jax.ShapeDtypeStruct((48,), jnp.float32),
        mesh=mesh,
        scratch_types=scratch,
        compiler_params=pltpu.CompilerParams(needs_layout_passes=False),
    )
    return fn(src, dst, cost, time, fast_pad, node_vec, rt_vec)


def _finish_body(coef_ref, pT_ref, p2T_ref, pp0_ref, pp1_ref, out_ref):
    r2 = jnp.maximum(pT_ref[2:3, :], 0.0)
    r3 = jnp.maximum(pT_ref[3:4, :], 0.0)
    dn = (((1,), (1,)), ((), ()))
    v0 = lax.dot_general(r2, pp0_ref[:, :], dn,
                         preferred_element_type=jnp.float32)
    v1 = lax.dot_general(r3, pp1_ref[:, :], dn,
                         preferred_element_type=jnp.float32)
    bmat = jnp.concatenate([
        pT_ref[4:5, :], pT_ref[5:6, :], v0, v1,
        p2T_ref[0:1, :], p2T_ref[1:2, :], p2T_ref[2:3, :], p2T_ref[3:4, :],
    ], axis=0)
    out3 = lax.dot_general(coef_ref[:, :], bmat, (((1,), (0,)), ((), ())),
                           preferred_element_type=jnp.float32)
    p1 = pT_ref[1:2, :]
    out_ref[:, :] = out3[0:1] + p1 * out3[1:2] + (p1 * p1) * out3[2:3]


def _finish(coef, params_p, params_p2, params_pp):
    pT = params_p.T
    p2T = jnp.stack([params_p2[:, 0, 0], params_p2[:, 1, 0],
                     params_p2[:, 0, 1], params_p2[:, 1, 1]], axis=0)
    pp0 = params_pp[:, :, 0]
    pp1 = params_pp[:, :, 1]
    out = pl.pallas_call(
        _finish_body,
        out_shape=jax.ShapeDtypeStruct((1, DIM), jnp.float32),
    )(coef, pT, p2T, pp0, pp1)
    return out[0]


def kernel(edge_index, edge_cost, edge_time, fastest_time, params_p,
           params_p2, params_pp, node, dest, remaining_time):
    src = edge_index[0]
    dst = edge_index[1]
    rt0 = jnp.float32(remaining_time)
    fast_pad = jnp.pad(fastest_time, (0, NPAD - N))
    node_vec = jnp.full((16,), node, jnp.int32)
    rt_vec = jnp.full((16,), rt0, jnp.float32)

    res = _sc_stats(src, dst, edge_cost, edge_time, fast_pad, node_vec,
                    rt_vec).reshape(3, 16)
    row0, row1, row2 = res[0], res[1], res[2]
    dnode = row0[8]
    g0 = jnp.stack([row0[0], rt0, row0[2], row0[3], row0[4], row0[5],
                    row0[6], row0[7]])
    s1 = jnp.stack([row1[0], row1[9], row1[2], row1[3], row1[4], row1[5],
                    row1[6], row1[7]]) / dnode
    s2 = row2[:8] / dnode
    coef = jnp.stack([g0, s1, s2], axis=0)
    return _finish(coef, params_p, params_p2, params_pp)


# speedup vs baseline: 64.8672x; 1.0557x over previous
import jax
import jax.numpy as jnp
from jax import lax
from jax.experimental import pallas as pl
from jax.experimental.pallas import tpu as pltpu
from jax.experimental.pallas import tpu_sc as plsc

N = 10000
E = 160000
DIM = 256
NPAD = 10240
NT = 16
EPT = E // NT
RPT = NPAD // NT
NV = RPT // 16
NCHUNK = EPT // 16
BIG = 1e30
NQ = 10


def _fill(ref, nvec, val, dtype):
    vec = jnp.full((16,), val, dtype)

    def body(j, c):
        ref[pl.ds(j * 16, 16)] = vec
        return c

    lax.fori_loop(0, nvec, body, 0)


def _merge(mbref, rbase, bb, out_tmp, is_min):
    init = BIG if is_min else 0.0

    def _half(h, first):
        pltpu.sync_copy(mbref.at[pl.ds(h, 8), pl.ds(rbase, RPT)], bb)

        def body(v, c):
            acc = bb[0, pl.ds(v * 16, 16)]
            for r in range(1, 8):
                x = bb[r, pl.ds(v * 16, 16)]
                acc = jnp.minimum(acc, x) if is_min else acc + x
            if first:
                out_tmp[pl.ds(v * 16, 16)] = acc
            else:
                prev = out_tmp[pl.ds(v * 16, 16)]
                out_tmp[pl.ds(v * 16, 16)] = (
                    jnp.minimum(prev, acc) if is_min else prev + acc)
            return c

        lax.fori_loop(0, NV, body, 0)

    del init
    _half(0, True)
    _half(8, False)


def _sc_body(src_hbm, dst_hbm, cost_hbm, time_hbm, fast_hbm, node_hbm,
             rt_hbm, out_hbm,
             src_loc, dst_loc, cost_loc, time_loc,
             w0, w1, w2, w3, w4,
             bb, tmp1, tmp2, tmp3,
             qb, acc1b, acc2b, nv, rv, st3,
             mb0, mb1,
             q0, q1, q2, q3, q4, q5, q6, q7, q8, q9, q10, q11,
             accsh, zc, bc):
    sid = lax.axis_index("s")
    cid = lax.axis_index("c")
    ebase = sid * EPT
    rbase = sid * RPT

    pltpu.sync_copy(src_hbm.at[pl.ds(ebase, EPT)], src_loc)
    pltpu.sync_copy(dst_hbm.at[pl.ds(ebase, EPT)], dst_loc)
    pltpu.sync_copy(cost_hbm.at[pl.ds(ebase, EPT)], cost_loc)
    pltpu.sync_copy(time_hbm.at[pl.ds(ebase, EPT)], time_loc)
    pltpu.sync_copy(node_hbm, nv)
    pltpu.sync_copy(rt_hbm, rv)
    nodesc = nv[:][0]
    rt0 = rv[:][0]

    _fill(tmp3, NV, 0.0, jnp.float32)
    pltpu.sync_copy(tmp3, zc.at[pl.ds(rbase, RPT)])
    _fill(tmp3, NV, BIG, jnp.float32)
    pltpu.sync_copy(tmp3, bc.at[pl.ds(rbase, RPT)])
    plsc.subcore_barrier()

    pltpu.sync_copy(zc, w0)
    pltpu.sync_copy(zc, w1)
    pltpu.sync_copy(zc, w2)
    pltpu.sync_copy(bc, w3)
    pltpu.sync_copy(bc, w4)
    ones16 = jnp.ones((16,), jnp.float32)

    def p1_chunk(b):
        srcv = src_loc[pl.ds(b, 16)]
        costv = cost_loc[pl.ds(b, 16)]
        timev = time_loc[pl.ds(b, 16)]
        cnt, lastm = plsc.scan_count(srcv)
        ndup = plsc.all_reduce_population_count(lastm)[0]

        @pl.when(ndup == 16)
        def _():
            plsc.addupdate_scatter(w0, [srcv], ones16)
            plsc.addupdate_scatter(w1, [srcv], costv)
            plsc.addupdate_scatter(w2, [srcv], timev)
            curc = plsc.load_gather(w3, [srcv])
            plsc.store_scatter(w3, [srcv], jnp.minimum(curc, costv))
            curt = plsc.load_gather(w4, [srcv])
            plsc.store_scatter(w4, [srcv], jnp.minimum(curt, timev))

        @pl.when(ndup < 16)
        def _():
            cntf = cnt.astype(jnp.float32)
            mn = jnp.min(cntf).astype(jnp.int32)
            mx = jnp.max(cntf).astype(jnp.int32)

            def rbody(r, c2):
                rm = cnt == r
                plsc.addupdate_scatter(w0, [srcv], ones16, mask=rm)
                plsc.addupdate_scatter(w1, [srcv], costv, mask=rm)
                plsc.addupdate_scatter(w2, [srcv], timev, mask=rm)
                curc = plsc.load_gather(w3, [srcv])
                plsc.store_scatter(w3, [srcv], jnp.minimum(curc, costv),
                                   mask=rm)
                curt = plsc.load_gather(w4, [srcv])
                plsc.store_scatter(w4, [srcv], jnp.minimum(curt, timev),
                                   mask=rm)
                return c2

            lax.fori_loop(mn, mx + 1, rbody, 0)

    def p1_group(i, c):
        for j in range(5):
            p1_chunk(i * 80 + 16 * j)
        return c

    lax.fori_loop(0, NCHUNK // 5, p1_group, 0)

    def ew(dst_ref, fn):
        def body(v, c):
            sl = pl.ds(v * 16, 16)
            dst_ref[sl] = fn(sl)
            return c

        lax.fori_loop(0, NV, body, 0)

    pltpu.sync_copy(w0, mb0.at[sid])
    pltpu.sync_copy(w1, mb1.at[sid])
    plsc.subcore_barrier()
    _merge(mb0, rbase, bb, tmp1, False)
    ew(tmp2, lambda sl: jnp.maximum(tmp1[sl], 1.0))
    pltpu.sync_copy(tmp1, q11.at[pl.ds(rbase, RPT)])
    pltpu.sync_copy(tmp2, q8.at[pl.ds(rbase, RPT)])
    _merge(mb1, rbase, bb, tmp3, False)
    ew(tmp3, lambda sl: tmp3[sl] / tmp2[sl])
    pltpu.sync_copy(tmp3, q2.at[pl.ds(rbase, RPT)])
    plsc.subcore_barrier()

    pltpu.sync_copy(w2, mb0.at[sid])
    pltpu.sync_copy(w3, mb1.at[sid])
    plsc.subcore_barrier()
    _merge(mb0, rbase, bb, tmp3, False)
    ew(tmp3, lambda sl: tmp3[sl] / tmp2[sl])
    pltpu.sync_copy(tmp3, q3.at[pl.ds(rbase, RPT)])
    _merge(mb1, rbase, bb, tmp3, True)
    ew(tmp3, lambda sl: jnp.where(tmp1[sl] > 0.0, tmp3[sl], 10000.0))
    pltpu.sync_copy(tmp3, q4.at[pl.ds(rbase, RPT)])
    plsc.subcore_barrier()

    pltpu.sync_copy(w4, mb0.at[sid])
    plsc.subcore_barrier()
    _merge(mb0, rbase, bb, tmp3, True)
    ew(tmp3, lambda sl: jnp.where(tmp1[sl] > 0.0, tmp3[sl], 10000.0))
    pltpu.sync_copy(tmp3, q7.at[pl.ds(rbase, RPT)])
    pltpu.sync_copy(fast_hbm.at[pl.ds(rbase, RPT)], tmp3)
    pltpu.sync_copy(tmp3, q0.at[pl.ds(rbase, RPT)])
    plsc.subcore_barrier()

    pltpu.sync_copy(q4, w0)
    pltpu.sync_copy(q7, w1)
    pltpu.sync_copy(bc, w2)
    pltpu.sync_copy(bc, w3)

    def p2_chunk(b):
        srcv = src_loc[pl.ds(b, 16)]
        costv = cost_loc[pl.ds(b, 16)]
        timev = time_loc[pl.ds(b, 16)]
        mc = plsc.load_gather(w0, [srcv])
        mt = plsc.load_gather(w1, [srcv])
        candt = jnp.where(costv <= mc, timev, BIG)
        candc = jnp.where(timev <= mt, costv, BIG)
        cnt, lastm = plsc.scan_count(srcv)
        ndup = plsc.all_reduce_population_count(lastm)[0]

        @pl.when(ndup == 16)
        def _():
            cur1 = plsc.load_gather(w2, [srcv])
            plsc.store_scatter(w2, [srcv], jnp.minimum(cur1, candt))
            cur2 = plsc.load_gather(w3, [srcv])
            plsc.store_scatter(w3, [srcv], jnp.minimum(cur2, candc))

        @pl.when(ndup < 16)
        def _():
            cntf = cnt.astype(jnp.float32)
            mn = jnp.min(cntf).astype(jnp.int32)
            mx = jnp.max(cntf).astype(jnp.int32)

            def rbody(r, c2):
                rm = cnt == r
                cur1 = plsc.load_gather(w2, [srcv])
                plsc.store_scatter(w2, [srcv], jnp.minimum(cur1, candt),
                                   mask=rm)
                cur2 = plsc.load_gather(w3, [srcv])
                plsc.store_scatter(w3, [srcv], jnp.minimum(cur2, candc),
                                   mask=rm)
                return c2

            lax.fori_loop(mn, mx + 1, rbody, 0)

    def p2_group(i, c):
        for j in range(5):
            p2_chunk(i * 80 + 16 * j)
        return c

    lax.fori_loop(0, NCHUNK // 5, p2_group, 0)
    pltpu.sync_copy(w2, mb0.at[sid])
    pltpu.sync_copy(w3, mb1.at[sid])
    plsc.subcore_barrier()
    pltpu.sync_copy(q11.at[pl.ds(rbase, RPT)], tmp1)
    _merge(mb0, rbase, bb, tmp3, True)
    ew(tmp3, lambda sl: jnp.where(tmp1[sl] > 0.0, tmp3[sl], 0.0))
    pltpu.sync_copy(tmp3, q5.at[pl.ds(rbase, RPT)])
    _merge(mb1, rbase, bb, tmp3, True)
    ew(tmp3, lambda sl: jnp.where(tmp1[sl] > 0.0, tmp3[sl], 0.0))
    pltpu.sync_copy(tmp3, q6.at[pl.ds(rbase, RPT)])
    plsc.subcore_barrier()

    pltpu.sync_copy(zc, w0)
    pltpu.sync_copy(zc, w1)

    def p3_chunk(b, m):
        dstv = dst_loc[pl.ds(b, 16)]
        timev = time_loc[pl.ds(b, 16)]
        mf = jnp.where(m, 1.0, 0.0)
        cnt, _ = plsc.scan_count(dstv)
        cntf = cnt.astype(jnp.float32)
        mn = jnp.min(cntf).astype(jnp.int32)
        mx = jnp.max(cntf).astype(jnp.int32)

        def rbody(r, c2):
            rm = cnt == r
            plsc.addupdate_scatter(w0, [dstv], mf, mask=rm)
            plsc.addupdate_scatter(w1, [dstv], mf * (rt0 - timev),
                                   mask=rm)
            return c2

        lax.fori_loop(mn, mx + 1, rbody, 0)

    def p3_body(i, c):
        b = i * 80
        ms = [src_loc[pl.ds(b + 16 * j, 16)] == nodesc for j in range(5)]
        mall = ms[0]
        for j in range(1, 5):
            mall = jnp.logical_or(mall, ms[j])
        anyv = plsc.all_reduce_population_count(mall)[0]

        @pl.when(anyv > 0)
        def _():
            for j in range(5):
                anyj = plsc.all_reduce_population_count(ms[j])[0]

                @pl.when(anyj > 0)
                def _(bj=b + 16 * j, mj=ms[j]):
                    p3_chunk(bj, mj)

        return c

    lax.fori_loop(0, NCHUNK // 5, p3_body, 0)
    pltpu.sync_copy(w0, mb0.at[sid])
    pltpu.sync_copy(w1, mb1.at[sid])
    plsc.subcore_barrier()
    pltpu.sync_copy(q8.at[pl.ds(rbase, RPT)], tmp2)
    _merge(mb0, rbase, bb, tmp1, False)
    ew(tmp3, lambda sl: tmp1[sl] / tmp2[sl])
    pltpu.sync_copy(tmp3, q10.at[pl.ds(rbase, RPT)])
    _merge(mb1, rbase, bb, tmp3, False)
    ew(tmp3, lambda sl: tmp3[sl] / jnp.maximum(tmp1[sl], 1.0))
    pltpu.sync_copy(tmp3, q9.at[pl.ds(rbase, RPT)])
    plsc.subcore_barrier()

    pltpu.sync_copy(q10, w2)
    pltpu.sync_copy(q9, w3)
    pltpu.sync_copy(zc, w0)
    pltpu.sync_copy(zc, w1)

    def p4_chunk(b, wv):
        srcv = src_loc[pl.ds(b, 16)]
        dstv = dst_loc[pl.ds(b, 16)]
        timev = time_loc[pl.ds(b, 16)]
        gf = jnp.where(wv > 0.0, 1.0, 0.0)
        t1 = plsc.load_gather(w3, [srcv])
        cnt, _ = plsc.scan_count(dstv)
        cntf = cnt.astype(jnp.float32)
        mn = jnp.min(cntf).astype(jnp.int32)
        mx = jnp.max(cntf).astype(jnp.int32)

        def rbody(r, c2):
            rm = cnt == r
            plsc.addupdate_scatter(w0, [dstv], gf, mask=rm)
            plsc.addupdate_scatter(w1, [dstv], gf * (t1 - timev),
                                   mask=rm)
            return c2

        lax.fori_loop(mn, mx + 1, rbody, 0)

    def p4_body(i, c):
        b = i * 80
        wvs = [plsc.load_gather(w2, [src_loc[pl.ds(b + 16 * j, 16)]])
               for j in range(5)]
        gs = [wv > 0.0 for wv in wvs]
        gall = gs[0]
        for j in range(1, 5):
            gall = jnp.logical_or(gall, gs[j])
        anyv = plsc.all_reduce_population_count(gall)[0]

        @pl.when(anyv > 0)
        def _():
            for j in range(5):
                anyj = plsc.all_reduce_population_count(gs[j])[0]

                @pl.when(anyj > 0)
                def _(bj=b + 16 * j, wvj=wvs[j]):
                    p4_chunk(bj, wvj)

        return c

    lax.fori_loop(0, NCHUNK // 5, p4_body, 0)
    pltpu.sync_copy(w0, mb0.at[sid])
    pltpu.sync_copy(w1, mb1.at[sid])
    plsc.subcore_barrier()
    _merge(mb0, rbase, bb, tmp1, False)
    _merge(mb1, rbase, bb, tmp3, False)
    ew(tmp3, lambda sl: tmp3[sl] / jnp.maximum(tmp1[sl], 1.0))
    pltpu.sync_copy(tmp3, q1.at[pl.ds(rbase, RPT)])
    plsc.subcore_barrier()

    z16 = jnp.zeros((16,), jnp.float32)
    for k in range(NQ):
        acc1b[pl.ds(k * 16, 16)] = z16
        acc2b[pl.ds(k * 16, 16)] = z16
    qrefs = (q0, q1, q2, q3, q4, q5, q6, q7, q8, q9)

    def p5_chunk(b, wv, mbm):
        mbf = jnp.where(mbm, 1.0, 0.0)
        dstv = dst_loc[pl.ds(b, 16)]
        for k in range(NQ):
            pltpu.sync_copy(qrefs[k].at[dstv], qb.at[pl.ds(k * 16, 16)])
        for k in range(NQ):
            sl = pl.ds(k * 16, 16)
            qv = qb[sl]
            acc2b[sl] = acc2b[sl] + wv * qv
            acc1b[sl] = acc1b[sl] + mbf * qv

    def p5_body(i, c):
        b = i * 80
        srcs = [src_loc[pl.ds(b + 16 * j, 16)] for j in range(5)]
        wvs = [plsc.load_gather(w2, [sv]) for sv in srcs]
        mbs = [sv == nodesc for sv in srcs]
        acts = [jnp.logical_or(wvs[j] > 0.0, mbs[j]) for j in range(5)]
        aall = acts[0]
        for j in range(1, 5):
            aall = jnp.logical_or(aall, acts[j])
        anyv = plsc.all_reduce_population_count(aall)[0]

        @pl.when(anyv > 0)
        def _():
            for j in range(5):
                anyj = plsc.all_reduce_population_count(acts[j])[0]

                @pl.when(anyj > 0)
                def _(bj=b + 16 * j, wvj=wvs[j], mbj=mbs[j]):
                    p5_chunk(bj, wvj, mbj)

        return c

    lax.fori_loop(0, NCHUNK // 5, p5_body, 0)
    pltpu.sync_copy(acc1b, accsh.at[pl.ds(sid * 2 * NQ * 16, NQ * 16)])
    pltpu.sync_copy(acc2b,
                    accsh.at[pl.ds(sid * 2 * NQ * 16 + NQ * 16, NQ * 16)])
    plsc.subcore_barrier()

    @pl.when(jnp.logical_and(sid == 0, cid == 0))
    def _():
        for k in range(NQ):
            acc1b[pl.ds(k * 16, 16)] = z16
            acc2b[pl.ds(k * 16, 16)] = z16
        for r in range(NT):
            pltpu.sync_copy(accsh.at[pl.ds(r * 2 * NQ * 16, NQ * 16)], qb)
            for k in range(NQ):
                sl = pl.ds(k * 16, 16)
                acc1b[sl] = acc1b[sl] + qb[sl]
            pltpu.sync_copy(
                accsh.at[pl.ds(r * 2 * NQ * 16 + NQ * 16, NQ * 16)], qb)
            for k in range(NQ):
                sl = pl.ds(k * 16, 16)
                acc2b[sl] = acc2b[sl] + qb[sl]
        lanes = lax.iota(jnp.int32, 16)
        s1l = []
        s2l = []
        for k in range(NQ):
            sl = pl.ds(k * 16, 16)
            s1l.append(jnp.sum(acc1b[sl]))
            s2l.append(jnp.sum(acc2b[sl]))
        nvec = jnp.full((16,), 0, jnp.int32) + nodesc
        row0 = jnp.zeros((16,), jnp.float32)
        for k in range(9):
            pltpu.sync_copy(qrefs[k].at[nvec], qb.at[pl.ds(k * 16, 16)])
            row0 = jnp.where(lanes == k, jnp.max(qb[pl.ds(k * 16, 16)]),
                             row0)
        dnv = qb[pl.ds(8 * 16, 16)]
        rinvv = jnp.ones((16,), jnp.float32) / dnv
        row0 = jnp.where(lanes == 1, rt0, row0)
        row1 = jnp.zeros((16,), jnp.float32)
        row2 = jnp.zeros((16,), jnp.float32)
        for t, k in enumerate([0, 9, 2, 3, 4, 5, 6, 7]):
            row1 = jnp.where(lanes == t, s1l[k], row1)
        for t in range(8):
            row2 = jnp.where(lanes == t, s2l[t], row2)
        row1 = row1 * rinvv
        row2 = row2 * rinvv
        st3[pl.ds(0, 16)] = row0
        st3[pl.ds(16, 16)] = row1
        st3[pl.ds(32, 16)] = row2
        pltpu.sync_copy(st3, out_hbm)


def _sc_stats(src, dst, cost, time, fast_pad, node_vec, rt_vec):
    mesh = plsc.VectorSubcoreMesh(core_axis_name="c", subcore_axis_name="s",
                                  num_cores=2, num_subcores=16)
    f32 = jnp.float32
    scratch = [
        pltpu.VMEM((EPT,), jnp.int32),
        pltpu.VMEM((EPT,), jnp.int32),
        pltpu.VMEM((EPT,), f32),
        pltpu.VMEM((EPT,), f32),
        pltpu.VMEM((NPAD,), f32),
        pltpu.VMEM((NPAD,), f32),
        pltpu.VMEM((NPAD,), f32),
        pltpu.VMEM((NPAD,), f32),
        pltpu.VMEM((NPAD,), f32),
        pltpu.VMEM((8, RPT), f32),
        pltpu.VMEM((RPT,), f32),
        pltpu.VMEM((RPT,), f32),
        pltpu.VMEM((RPT,), f32),
        pltpu.VMEM((NQ * 16,), f32),
        pltpu.VMEM((NQ * 16,), f32),
        pltpu.VMEM((NQ * 16,), f32),
        pltpu.VMEM((16,), jnp.int32),
        pltpu.VMEM((16,), f32),
        pltpu.VMEM((48,), f32),
        pltpu.VMEM_SHARED((NT, NPAD), f32),
        pltpu.VMEM_SHARED((NT, NPAD), f32),
    ] + [pltpu.VMEM_SHARED((NPAD,), f32) for _ in range(12)] + [
        pltpu.VMEM_SHARED((NT * 2 * NQ * 16,), f32),
        pltpu.VMEM_SHARED((NPAD,), f32),
        pltpu.VMEM_SHARED((NPAD,), f32),
    ]
    fn = pl.kernel(
        _sc_body,
        out_type=jax.ShapeDtypeStruct((48,), jnp.float32),
        mesh=mesh,
        scratch_types=scratch,
        compiler_params=pltpu.CompilerParams(needs_layout_passes=False),
    )
    return fn(src, dst, cost, time, fast_pad, node_vec, rt_vec)


def _finish_body(coef_ref, pT_ref, p2T_ref, pp0_ref, pp1_ref, out_ref):
    r2 = jnp.maximum(pT_ref[2:3, :], 0.0)
    r3 = jnp.maximum(pT_ref[3:4, :], 0.0)
    dn = (((1,), (1,)), ((), ()))
    v0 = lax.dot_general(r2, pp0_ref[:, :], dn,
                         preferred_element_type=jnp.float32)
    v1 = lax.dot_general(r3, pp1_ref[:, :], dn,
                         preferred_element_type=jnp.float32)
    bmat = jnp.concatenate([
        pT_ref[4:5, :], pT_ref[5:6, :], v0, v1,
        p2T_ref[0:1, :], p2T_ref[1:2, :], p2T_ref[2:3, :], p2T_ref[3:4, :],
    ], axis=0)
    out3 = lax.dot_general(coef_ref[:, :], bmat, (((1,), (0,)), ((), ())),
                           preferred_element_type=jnp.float32)
    p1 = pT_ref[1:2, :]
    out_ref[:, :] = out3[0:1] + p1 * out3[1:2] + (p1 * p1) * out3[2:3]


def _finish(coef, params_p, params_p2, params_pp):
    pT = params_p.T
    p2T = jnp.stack([params_p2[:, 0, 0], params_p2[:, 1, 0],
                     params_p2[:, 0, 1], params_p2[:, 1, 1]], axis=0)
    pp0 = params_pp[:, :, 0]
    pp1 = params_pp[:, :, 1]
    out = pl.pallas_call(
        _finish_body,
        out_shape=jax.ShapeDtypeStruct((1, DIM), jnp.float32),
    )(coef, pT, p2T, pp0, pp1)
    return out[0]


def kernel(edge_index, edge_cost, edge_time, fastest_time, params_p,
           params_p2, params_pp, node, dest, remaining_time):
    src = edge_index[0]
    dst = edge_index[1]
    rt0 = jnp.float32(remaining_time)
    fast_pad = jnp.pad(fastest_time, (0, NPAD - N))
    node_vec = jnp.full((16,), node, jnp.int32)
    rt_vec = jnp.full((16,), rt0, jnp.float32)

    res = _sc_stats(src, dst, edge_cost, edge_time, fast_pad, node_vec,
                    rt_vec).reshape(3, 16)
    coef = res[:, :8]
    return _finish(coef, params_p, params_p2, params_pp)
